# baseline (speedup 1.0000x reference)
import functools

import jax
import jax.numpy as jnp
from jax import lax
from jax.experimental import pallas as pl
from jax.experimental.pallas import tpu as pltpu
from jax.experimental.pallas import tpu_sc as plsc

_NEG_SLOPE = 0.01
_NUM_CORES = 2
_NUM_SUBCORES = 16
_NUM_WORKERS = _NUM_CORES * _NUM_SUBCORES
_NBUF = 2
_SLABS = 16


def _proj_body(t_ref, w_ref, b_ref, o_ref):
    y = jnp.dot(t_ref[...], w_ref[...], preferred_element_type=jnp.float32)
    y = y + b_ref[...]
    o_ref[...] = jnp.where(y >= 0, y, _NEG_SLOPE * y).astype(o_ref.dtype)


def _project(table, W, b):
    rows, e = table.shape
    o = W.shape[1]
    w_pad = jnp.pad(W, ((0, 0), (0, 128 - o)))
    b_pad = jnp.pad(b.reshape(1, o), ((0, 0), (0, 128 - o)))
    blk = 8
    for cand in range(8, 12501, 8):
        if rows % cand == 0:
            blk = cand
    return pl.pallas_call(
        _proj_body,
        grid=(rows // blk,),
        in_specs=[
            pl.BlockSpec((blk, e), lambda i: (i, 0)),
            pl.BlockSpec((e, 128), lambda i: (0, 0)),
            pl.BlockSpec((1, 128), lambda i: (0, 0)),
        ],
        out_specs=pl.BlockSpec((blk, 128), lambda i: (i, 0)),
        out_shape=jax.ShapeDtypeStruct((rows, 128), jnp.float32),
    )(table, w_pad, b_pad)


def _gather_rows(z, idx_flat, fields):
    n_idx = idx_flat.shape[0]
    d = z.shape[1]
    batch = n_idx // fields
    chunk = _SLABS * fields
    assert n_idx % (_NUM_WORKERS * _NBUF * chunk) == 0
    b_per_w = n_idx // _NUM_WORKERS
    batches_per_w = batch // _NUM_WORKERS
    nchunk = b_per_w // chunk
    mesh = plsc.VectorSubcoreMesh(core_axis_name="core",
                                  subcore_axis_name="subcore")

    @functools.partial(
        pl.kernel,
        mesh=mesh,
        out_type=jax.ShapeDtypeStruct((batch, fields, d), z.dtype),
        scratch_types=[
            pltpu.VMEM((b_per_w,), jnp.int32),
            pltpu.VMEM((_NBUF, chunk, d), z.dtype),
        ] + [pltpu.SemaphoreType.DMA] * (2 * _NBUF),
    )
    def k(z_hbm, i_hbm, o_hbm, idx_v, rows_v, *sems):
        gsems, wsems = sems[:_NBUF], sems[_NBUF:]
        wid = lax.axis_index("subcore") * _NUM_CORES + lax.axis_index("core")
        base = wid * b_per_w
        bbase = wid * batches_per_w
        pltpu.sync_copy(i_hbm.at[pl.ds(base, b_per_w)], idx_v)

        def gather(c, j):
            return pltpu.make_async_copy(
                z_hbm.at[idx_v.at[pl.ds(c * chunk, chunk)]],
                rows_v.at[j], gsems[j])

        def write_slab(c, j, i):
            return pltpu.make_async_copy(
                rows_v.at[j, pl.ds(i * fields, fields)],
                o_hbm.at[bbase + c * _SLABS + i], wsems[j])

        def write_start(c, j):
            for i in range(_SLABS):
                write_slab(c, j, i).start()

        def write_wait(c, j):
            for i in range(_SLABS):
                write_slab(c, j, i).wait()

        for j in range(_NBUF):
            gather(j, j).start()

        @pl.loop(0, nchunk - _NBUF, step=_NBUF)
        def _(c0):
            for j in range(_NBUF):
                gather(c0 + j, j).wait()
                write_start(c0 + j, j)
            for j in range(_NBUF):
                write_wait(c0 + j, j)
                gather(c0 + _NBUF + j, j).start()

        c0 = nchunk - _NBUF
        for j in range(_NBUF):
            gather(c0 + j, j).wait()
            write_start(c0 + j, j)
        for j in range(_NBUF):
            write_wait(c0 + j, j)

    return k(z, idx_flat)


def kernel(feature, table, W, b):
    batch, fields = feature.shape
    out_dim = W.shape[1]
    z = _project(table, W, b)
    idx_flat = feature.reshape(-1).astype(jnp.int32)
    out3 = _gather_rows(z, idx_flat, fields)
    return out3[:, :, :out_dim] if out3.shape[-1] != out_dim else out3


# speedup vs baseline: 1.0148x; 1.0148x over previous
import functools

import jax
import jax.numpy as jnp
from jax import lax
from jax.experimental import pallas as pl
from jax.experimental.pallas import tpu as pltpu
from jax.experimental.pallas import tpu_sc as plsc

_NEG_SLOPE = 0.01
_NUM_CORES = 2
_NUM_SUBCORES = 16
_NUM_WORKERS = _NUM_CORES * _NUM_SUBCORES
_NBUF = 4
_SLABS = 8


def _proj_body(t_ref, w_ref, b_ref, o_ref):
    y = jnp.dot(t_ref[...], w_ref[...], preferred_element_type=jnp.float32)
    y = y + b_ref[...]
    o_ref[...] = jnp.where(y >= 0, y, _NEG_SLOPE * y).astype(o_ref.dtype)


def _project(table, W, b):
    rows, e = table.shape
    o = W.shape[1]
    w_pad = jnp.pad(W, ((0, 0), (0, 128 - o)))
    b_pad = jnp.pad(b.reshape(1, o), ((0, 0), (0, 128 - o)))
    blk = 8
    for cand in range(8, 12501, 8):
        if rows % cand == 0:
            blk = cand
    return pl.pallas_call(
        _proj_body,
        grid=(rows // blk,),
        in_specs=[
            pl.BlockSpec((blk, e), lambda i: (i, 0)),
            pl.BlockSpec((e, 128), lambda i: (0, 0)),
            pl.BlockSpec((1, 128), lambda i: (0, 0)),
        ],
        out_specs=pl.BlockSpec((blk, 128), lambda i: (i, 0)),
        out_shape=jax.ShapeDtypeStruct((rows, 128), jnp.float32),
    )(table, w_pad, b_pad)


def _gather_rows(z, idx_flat, fields):
    n_idx = idx_flat.shape[0]
    d = z.shape[1]
    batch = n_idx // fields
    chunk = _SLABS * fields
    assert n_idx % (_NUM_WORKERS * _NBUF * chunk) == 0
    b_per_w = n_idx // _NUM_WORKERS
    batches_per_w = batch // _NUM_WORKERS
    nchunk = b_per_w // chunk
    mesh = plsc.VectorSubcoreMesh(core_axis_name="core",
                                  subcore_axis_name="subcore")

    @functools.partial(
        pl.kernel,
        mesh=mesh,
        out_type=jax.ShapeDtypeStruct((batch, fields, d), z.dtype),
        scratch_types=[
            pltpu.VMEM((b_per_w,), jnp.int32),
            pltpu.VMEM((_NBUF, chunk, d), z.dtype),
        ] + [pltpu.SemaphoreType.DMA] * (2 * _NBUF),
    )
    def k(z_hbm, i_hbm, o_hbm, idx_v, rows_v, *sems):
        gsems, wsems = sems[:_NBUF], sems[_NBUF:]
        wid = lax.axis_index("subcore") * _NUM_CORES + lax.axis_index("core")
        base = wid * b_per_w
        bbase = wid * batches_per_w
        pltpu.sync_copy(i_hbm.at[pl.ds(base, b_per_w)], idx_v)

        def gather(c, j):
            return pltpu.make_async_copy(
                z_hbm.at[idx_v.at[pl.ds(c * chunk, chunk)]],
                rows_v.at[j], gsems[j])

        def write_slab(c, j, i):
            return pltpu.make_async_copy(
                rows_v.at[j, pl.ds(i * fields, fields)],
                o_hbm.at[bbase + c * _SLABS + i], wsems[j])

        def write_start(c, j):
            for i in range(_SLABS):
                write_slab(c, j, i).start()

        def write_wait(c, j):
            for i in range(_SLABS):
                write_slab(c, j, i).wait()

        for j in range(_NBUF):
            gather(j, j).start()

        @pl.loop(0, nchunk - _NBUF, step=_NBUF)
        def _(c0):
            for j in range(_NBUF):
                gather(c0 + j, j).wait()
                write_start(c0 + j, j)
            for j in range(_NBUF):
                write_wait(c0 + j, j)
                gather(c0 + _NBUF + j, j).start()

        c0 = nchunk - _NBUF
        for j in range(_NBUF):
            gather(c0 + j, j).wait()
            write_start(c0 + j, j)
        for j in range(_NBUF):
            write_wait(c0 + j, j)

    return k(z, idx_flat)


def kernel(feature, table, W, b):
    batch, fields = feature.shape
    out_dim = W.shape[1]
    z = _project(table, W, b)
    idx_flat = feature.reshape(-1).astype(jnp.int32)
    out3 = _gather_rows(z, idx_flat, fields)
    return out3[:, :, :out_dim] if out3.shape[-1] != out_dim else out3
